# baseline (speedup 1.0000x reference)
import functools

import jax
import jax.numpy as jnp
from jax import lax
from jax.experimental import pallas as pl
from jax.experimental.pallas import tpu as pltpu
from jax.experimental.pallas import tpu_sc as plsc

B = 4096
D = 44
DP = 48
H = 1024
E = 8
A = 8
RH = H // 2
BB = 1024
NB = B // BB
BM = 256
NBLK = (B * 2) // BM + E
P = NBLK * BM
NW = 32
TPW = B // NW


def _ln(x, g, b, eps=1e-5):
    m = jnp.mean(x, axis=-1, keepdims=True)
    v = jnp.mean((x - m) * (x - m), axis=-1, keepdims=True)
    return (x - m) * jax.lax.rsqrt(v + eps) * g + b


def _router_body(state_ref, stateb_ref, rw1_ref, rb1_ref, rw2_ref, rb2_ref,
                 vw1_ref, vb1_ref, vg_ref, vbe_ref,
                 vw2_ref, vb2_ref, vw3_ref, vb3_ref,
                 probs_ref, g0_ref, g1_ref, pos_ref, blk_ref, value_ref):
    b = pl.program_id(0)

    @pl.when(b == 0)
    def _router_meta():
        x = state_ref[...]
        h = jnp.maximum(jnp.dot(x, rw1_ref[...],
                                preferred_element_type=jnp.float32)
                        + rb1_ref[...], 0.0)
        logits = jnp.dot(h, rw2_ref[...],
                         preferred_element_type=jnp.float32) + rb2_ref[...]
        mx = jnp.max(logits, axis=-1, keepdims=True)
        ex = jnp.exp(logits - mx)
        probs = ex / jnp.sum(ex, axis=-1, keepdims=True)
        probs_ref[...] = probs
        p1 = jnp.max(probs, axis=-1, keepdims=True)
        i1 = jnp.argmax(probs, axis=-1)[:, None]
        eids = jax.lax.broadcasted_iota(jnp.int32, (B, E), 1)
        masked = jnp.where(eids == i1, -jnp.inf, probs)
        p2 = jnp.max(masked, axis=-1, keepdims=True)
        i2 = jnp.argmax(masked, axis=-1)[:, None]
        denom = p1 + p2 + 1e-8
        g0_ref[...] = p1 / denom
        g1_ref[...] = p2 / denom
        sel1 = (eids == i1).astype(jnp.float32)
        sel2 = (eids == i2).astype(jnp.float32)
        sel = sel1 + sel2
        rr = jax.lax.broadcasted_iota(jnp.int32, (512, 512), 0)
        cc = jax.lax.broadcasted_iota(jnp.int32, (512, 512), 1)
        tri = (cc < rr).astype(jnp.float32)
        carry = jnp.zeros((1, E), jnp.float32)
        chunks = []
        for c in range(B // 512):
            ch = sel[c * 512:(c + 1) * 512, :]
            chunks.append(jnp.dot(tri, ch,
                                  preferred_element_type=jnp.float32)
                          + carry)
            carry = carry + jnp.sum(ch, axis=0, keepdims=True)
        rank = jnp.concatenate(chunks, axis=0)
        counts = carry
        nblk_e = jnp.ceil(counts / BM)
        u_r = jax.lax.broadcasted_iota(jnp.int32, (E, E), 0)
        u_c = jax.lax.broadcasted_iota(jnp.int32, (E, E), 1)
        upper = (u_r < u_c).astype(jnp.float32)
        cumblk_excl = jnp.dot(nblk_e, upper,
                              preferred_element_type=jnp.float32)
        base = cumblk_excl * BM
        rank1 = jnp.sum(sel1 * rank, axis=-1, keepdims=True)
        rank2 = jnp.sum(sel2 * (rank + sel1), axis=-1, keepdims=True)
        base1 = jnp.sum(sel1 * base, axis=-1, keepdims=True)
        base2 = jnp.sum(sel2 * base, axis=-1, keepdims=True)
        pos_ref[...] = jnp.concatenate(
            [base1 + rank1, base2 + rank2], axis=1).astype(jnp.int32)
        cumblk_incl = cumblk_excl + nblk_e
        biota = jax.lax.broadcasted_iota(jnp.int32, (NBLK, E), 0)
        blk = jnp.sum((biota >= cumblk_incl.astype(jnp.int32))
                      .astype(jnp.int32), axis=-1, keepdims=True)
        blk_ref[...] = jnp.minimum(blk, E - 1)

    xb = stateb_ref[...].astype(jnp.bfloat16)
    v = jnp.maximum(
        _ln(jnp.dot(xb, vw1_ref[...], preferred_element_type=jnp.float32)
            + vb1_ref[...], vg_ref[...], vbe_ref[...]), 0.0)
    v = jnp.maximum(jnp.dot(v.astype(jnp.bfloat16), vw2_ref[...],
                            preferred_element_type=jnp.float32)
                    + vb2_ref[...], 0.0)
    value_ref[...] = jnp.dot(v.astype(jnp.bfloat16), vw3_ref[...],
                             preferred_element_type=jnp.float32) + vb3_ref[...]


def _router_call(state, p):
    full = lambda shape: pl.BlockSpec(shape, lambda b: (0,) * len(shape))
    per_b = lambda shape: pl.BlockSpec(
        shape, lambda b: (b,) + (0,) * (len(shape) - 1))
    return pl.pallas_call(
        _router_body,
        grid=(NB,),
        in_specs=[
            full((B, D)), per_b((BB, D)),
            full((D, RH)), full((RH,)), full((RH, E)), full((E,)),
            full((D, H)), full((H,)), full((H,)), full((H,)),
            full((H, RH)), full((RH,)), full((RH, 1)), full((1,)),
        ],
        out_specs=[
            full((B, E)), full((B, 1)), full((B, 1)),
            full((B, 2)), full((NBLK, 1)), per_b((BB, 1)),
        ],
        out_shape=[
            jax.ShapeDtypeStruct((B, E), jnp.float32),
            jax.ShapeDtypeStruct((B, 1), jnp.float32),
            jax.ShapeDtypeStruct((B, 1), jnp.float32),
            jax.ShapeDtypeStruct((B, 2), jnp.int32),
            jax.ShapeDtypeStruct((NBLK, 1), jnp.int32),
            jax.ShapeDtypeStruct((B, 1), jnp.float32),
        ],
        compiler_params=pltpu.CompilerParams(
            dimension_semantics=("arbitrary",),
        ),
    )(state, state,
      p['router_w1'], p['router_b1'], p['router_w2'], p['router_b2'],
      p['val_w1'].astype(jnp.bfloat16), p['val_b1'], p['val_g'],
      p['val_be'],
      p['val_w2'].astype(jnp.bfloat16), p['val_b2'],
      p['val_w3'].astype(jnp.bfloat16), p['val_b3'])


def _sc_dispatch_body(state_hbm, pos_hbm, out_hbm, idx_v, rows_v, sem):
    wid = lax.axis_index("s") * 2 + lax.axis_index("c")
    base = wid * TPW
    pltpu.sync_copy(state_hbm.at[pl.ds(base, TPW)], rows_v)
    for k in range(2):
        pltpu.sync_copy(pos_hbm.at[k, wid], idx_v)
        pltpu.async_copy(rows_v, out_hbm.at[idx_v], sem).wait()


def _sc_dispatch(state_pad, pos_sc):
    f = functools.partial(
        pl.kernel,
        out_type=jax.ShapeDtypeStruct((P, DP), jnp.float32),
        mesh=plsc.VectorSubcoreMesh(core_axis_name="c",
                                    subcore_axis_name="s"),
        scratch_types=[
            pltpu.VMEM((TPW,), jnp.int32),
            pltpu.VMEM((TPW, DP), jnp.float32),
            pltpu.SemaphoreType.DMA,
        ],
        compiler_params=pltpu.CompilerParams(use_tc_tiling_on_sc=False),
    )(_sc_dispatch_body)
    return f(state_pad, pos_sc)


def _ffn_body(m_ref, x_ref, w1_ref, b1_ref, g1_ref, be1_ref,
              w2_ref, b2_ref, g2_ref, be2_ref, w3_ref, b3_ref, out_ref):
    x = x_ref[...].astype(jnp.bfloat16)
    h1 = jnp.dot(x, w1_ref[0], preferred_element_type=jnp.float32) \
        + b1_ref[0]
    h1 = jnp.maximum(_ln(h1, g1_ref[0, 0], be1_ref[0, 0]), 0.0)
    h2 = jnp.dot(h1.astype(jnp.bfloat16), w2_ref[0],
                 preferred_element_type=jnp.float32) + b2_ref[0]
    h2 = jnp.maximum(_ln(h2, g2_ref[0, 0], be2_ref[0, 0]), 0.0)
    out_ref[...] = jnp.dot(h2.astype(jnp.bfloat16), w3_ref[0],
                           preferred_element_type=jnp.float32) + b3_ref[0]


def _ffn_call(blk, dispatch, p, ew1p):
    per_e = lambda shape: pl.BlockSpec(
        (1,) + shape, lambda i, m: (m[i], 0, 0))
    grid_spec = pltpu.PrefetchScalarGridSpec(
        num_scalar_prefetch=1,
        grid=(NBLK,),
        in_specs=[
            pl.BlockSpec((BM, DP), lambda i, m: (i, 0)),
            per_e((DP, H)), per_e((1, H)), per_e((1, H)), per_e((1, H)),
            per_e((H, H)), per_e((1, H)), per_e((1, H)), per_e((1, H)),
            per_e((H, 2 * A)), per_e((1, 2 * A)),
        ],
        out_specs=pl.BlockSpec((BM, 2 * A), lambda i, m: (i, 0)),
    )
    return pl.pallas_call(
        _ffn_body,
        grid_spec=grid_spec,
        out_shape=jax.ShapeDtypeStruct((P, 2 * A), jnp.float32),
        compiler_params=pltpu.CompilerParams(
            dimension_semantics=("arbitrary",),
        ),
    )(blk, dispatch,
      ew1p.astype(jnp.bfloat16), p['exp_b1'][:, None, :],
      p['exp_g1'][:, None, :], p['exp_be1'][:, None, :],
      p['exp_w2'].astype(jnp.bfloat16), p['exp_b2'][:, None, :],
      p['exp_g2'][:, None, :], p['exp_be2'][:, None, :],
      p['exp_w3'].astype(jnp.bfloat16), p['exp_b3'][:, None, :])


def _sc_gather_body(ffn_hbm, pos_hbm, r0_hbm, r1_hbm, idx_v, rows_v, sem):
    wid = lax.axis_index("s") * 2 + lax.axis_index("c")
    base = wid * TPW
    for k, out in enumerate((r0_hbm, r1_hbm)):
        pltpu.sync_copy(pos_hbm.at[k, wid], idx_v)
        pltpu.async_copy(ffn_hbm.at[idx_v], rows_v, sem).wait()
        pltpu.sync_copy(rows_v, out.at[pl.ds(base, TPW)])


def _sc_gather(ffn_out, pos_sc):
    f = functools.partial(
        pl.kernel,
        out_type=(jax.ShapeDtypeStruct((B, 2 * A), jnp.float32),
                  jax.ShapeDtypeStruct((B, 2 * A), jnp.float32)),
        mesh=plsc.VectorSubcoreMesh(core_axis_name="c",
                                    subcore_axis_name="s"),
        scratch_types=[
            pltpu.VMEM((TPW,), jnp.int32),
            pltpu.VMEM((TPW, 2 * A), jnp.float32),
            pltpu.SemaphoreType.DMA,
        ],
        compiler_params=pltpu.CompilerParams(use_tc_tiling_on_sc=False),
    )(_sc_gather_body)
    return f(ffn_out, pos_sc)


def _combine_body(r0_ref, r1_ref, g0_ref, g1_ref, alpha_ref, beta_ref):
    acc = g0_ref[...] * r0_ref[...] + g1_ref[...] * r1_ref[...]
    alpha_ref[...] = jax.nn.softplus(acc[:, :A]) + 1.0
    beta_ref[...] = jax.nn.softplus(acc[:, A:]) + 1.0


def _combine_call(r0, r1, g0, g1):
    return pl.pallas_call(
        _combine_body,
        out_shape=[jax.ShapeDtypeStruct((B, A), jnp.float32),
                   jax.ShapeDtypeStruct((B, A), jnp.float32)],
    )(r0, r1, g0, g1)


def kernel(state, params):
    p = params
    state_pad = jnp.pad(state, ((0, 0), (0, DP - D)))
    ew1p = jnp.pad(p['exp_w1'], ((0, 0), (0, DP - D), (0, 0)))
    probs, g0, g1, pos, blk, value = _router_call(state, p)
    pos_sc = pos.T.reshape(2, NW, TPW)
    dispatch = _sc_dispatch(state_pad, pos_sc)
    ffn_out = _ffn_call(blk.reshape(NBLK), dispatch, p, ew1p)
    r0, r1 = _sc_gather(ffn_out, pos_sc)
    alpha, beta = _combine_call(r0, r1, g0, g1)
    return (alpha, beta, value, probs)


# speedup vs baseline: 1.0808x; 1.0808x over previous
import functools

import jax
import jax.numpy as jnp
from jax import lax
from jax.experimental import pallas as pl
from jax.experimental.pallas import tpu as pltpu
from jax.experimental.pallas import tpu_sc as plsc

B = 4096
D = 44
DP = 48
H = 1024
E = 8
A = 8
RH = H // 2
BB = 1024
NB = B // BB
BM = 256
NBLK = (B * 2) // BM + E
P = NBLK * BM
NW = 32
TPW = B // NW


def _ln(x, g, b, eps=1e-5):
    m = jnp.mean(x, axis=-1, keepdims=True)
    v = jnp.mean((x - m) * (x - m), axis=-1, keepdims=True)
    return (x - m) * jax.lax.rsqrt(v + eps) * g + b


def _router_body(state_ref, stateb_ref, rw1_ref, rb1_ref, rw2_ref, rb2_ref,
                 vw1_ref, vb1_ref, vg_ref, vbe_ref,
                 vw2_ref, vb2_ref, vw3_ref, vb3_ref,
                 probs_ref, g0_ref, g1_ref, pos_ref, blk_ref, value_ref):
    b = pl.program_id(0)

    @pl.when(b == 0)
    def _router_meta():
        x = state_ref[...]
        h = jnp.maximum(jnp.dot(x, rw1_ref[...],
                                preferred_element_type=jnp.float32)
                        + rb1_ref[...], 0.0)
        logits = jnp.dot(h, rw2_ref[...],
                         preferred_element_type=jnp.float32) + rb2_ref[...]
        mx = jnp.max(logits, axis=-1, keepdims=True)
        ex = jnp.exp(logits - mx)
        probs = ex / jnp.sum(ex, axis=-1, keepdims=True)
        probs_ref[...] = probs
        p1 = jnp.max(probs, axis=-1, keepdims=True)
        i1 = jnp.argmax(probs, axis=-1)[:, None]
        eids = jax.lax.broadcasted_iota(jnp.int32, (B, E), 1)
        masked = jnp.where(eids == i1, -jnp.inf, probs)
        p2 = jnp.max(masked, axis=-1, keepdims=True)
        i2 = jnp.argmax(masked, axis=-1)[:, None]
        denom = p1 + p2 + 1e-8
        g0_ref[...] = p1 / denom
        g1_ref[...] = p2 / denom
        sel1 = (eids == i1).astype(jnp.float32)
        sel2 = (eids == i2).astype(jnp.float32)
        sel = sel1 + sel2
        rr = jax.lax.broadcasted_iota(jnp.int32, (512, 512), 0)
        cc = jax.lax.broadcasted_iota(jnp.int32, (512, 512), 1)
        tri = (cc < rr).astype(jnp.float32)
        carry = jnp.zeros((1, E), jnp.float32)
        chunks = []
        for c in range(B // 512):
            ch = sel[c * 512:(c + 1) * 512, :]
            chunks.append(jnp.dot(tri, ch,
                                  preferred_element_type=jnp.float32)
                          + carry)
            carry = carry + jnp.sum(ch, axis=0, keepdims=True)
        rank = jnp.concatenate(chunks, axis=0)
        counts = carry
        nblk_e = jnp.ceil(counts / BM)
        u_r = jax.lax.broadcasted_iota(jnp.int32, (E, E), 0)
        u_c = jax.lax.broadcasted_iota(jnp.int32, (E, E), 1)
        upper = (u_r < u_c).astype(jnp.float32)
        cumblk_excl = jnp.dot(nblk_e, upper,
                              preferred_element_type=jnp.float32)
        base = cumblk_excl * BM
        rank1 = jnp.sum(sel1 * rank, axis=-1, keepdims=True)
        rank2 = jnp.sum(sel2 * (rank + sel1), axis=-1, keepdims=True)
        base1 = jnp.sum(sel1 * base, axis=-1, keepdims=True)
        base2 = jnp.sum(sel2 * base, axis=-1, keepdims=True)
        pos_ref[...] = jnp.concatenate(
            [base1 + rank1, base2 + rank2], axis=1).astype(jnp.int32)
        cumblk_incl = cumblk_excl + nblk_e
        biota = jax.lax.broadcasted_iota(jnp.int32, (NBLK, E), 0)
        blk = jnp.sum((biota >= cumblk_incl.astype(jnp.int32))
                      .astype(jnp.int32), axis=-1, keepdims=True)
        blk_ref[...] = jnp.minimum(blk, E - 1)

    bf = jnp.bfloat16
    xb = stateb_ref[...].astype(bf)
    v = jnp.maximum(
        _ln(jnp.dot(xb, vw1_ref[...].astype(bf),
                    preferred_element_type=jnp.float32)
            + vb1_ref[...], vg_ref[...], vbe_ref[...]), 0.0)
    v = jnp.maximum(jnp.dot(v.astype(bf), vw2_ref[...].astype(bf),
                            preferred_element_type=jnp.float32)
                    + vb2_ref[...], 0.0)
    value_ref[...] = jnp.dot(v.astype(bf), vw3_ref[...].astype(bf),
                             preferred_element_type=jnp.float32) + vb3_ref[...]


def _router_call(state, p):
    full = lambda shape: pl.BlockSpec(shape, lambda b: (0,) * len(shape))
    per_b = lambda shape: pl.BlockSpec(
        shape, lambda b: (b,) + (0,) * (len(shape) - 1))
    return pl.pallas_call(
        _router_body,
        grid=(NB,),
        in_specs=[
            full((B, D)), per_b((BB, D)),
            full((D, RH)), full((RH,)), full((RH, E)), full((E,)),
            full((D, H)), full((H,)), full((H,)), full((H,)),
            full((H, RH)), full((RH,)), full((RH, 1)), full((1,)),
        ],
        out_specs=[
            full((B, E)), full((B, 1)), full((B, 1)),
            full((B, 2)), full((NBLK, 1)), per_b((BB, 1)),
        ],
        out_shape=[
            jax.ShapeDtypeStruct((B, E), jnp.float32),
            jax.ShapeDtypeStruct((B, 1), jnp.float32),
            jax.ShapeDtypeStruct((B, 1), jnp.float32),
            jax.ShapeDtypeStruct((B, 2), jnp.int32),
            jax.ShapeDtypeStruct((NBLK, 1), jnp.int32),
            jax.ShapeDtypeStruct((B, 1), jnp.float32),
        ],
        compiler_params=pltpu.CompilerParams(
            dimension_semantics=("arbitrary",),
        ),
    )(state, state,
      p['router_w1'], p['router_b1'], p['router_w2'], p['router_b2'],
      p['val_w1'], p['val_b1'], p['val_g'], p['val_be'],
      p['val_w2'], p['val_b2'], p['val_w3'], p['val_b3'])


def _sc_dispatch_body(state_hbm, pos_hbm, out_hbm, idx_v, rows_v, sem):
    wid = lax.axis_index("s") * 2 + lax.axis_index("c")
    base = wid * TPW
    pltpu.sync_copy(state_hbm.at[pl.ds(base, TPW)], rows_v)
    for k in range(2):
        pltpu.sync_copy(pos_hbm.at[k, wid], idx_v)
        pltpu.async_copy(rows_v, out_hbm.at[idx_v], sem).wait()


def _sc_dispatch(state_pad, pos_sc):
    f = functools.partial(
        pl.kernel,
        out_type=jax.ShapeDtypeStruct((P, DP), jnp.float32),
        mesh=plsc.VectorSubcoreMesh(core_axis_name="c",
                                    subcore_axis_name="s"),
        scratch_types=[
            pltpu.VMEM((TPW,), jnp.int32),
            pltpu.VMEM((TPW, DP), jnp.float32),
            pltpu.SemaphoreType.DMA,
        ],
        compiler_params=pltpu.CompilerParams(use_tc_tiling_on_sc=False),
    )(_sc_dispatch_body)
    return f(state_pad, pos_sc)


def _ffn_body(m_ref, x_ref, w1_ref, b1_ref, g1_ref, be1_ref,
              w2_ref, b2_ref, g2_ref, be2_ref, w3_ref, b3_ref, out_ref):
    bf = jnp.bfloat16
    x = x_ref[...].astype(bf)
    h1 = jnp.dot(x, w1_ref[0].astype(bf),
                 preferred_element_type=jnp.float32) + b1_ref[0]
    h1 = jnp.maximum(_ln(h1, g1_ref[0, 0], be1_ref[0, 0]), 0.0)
    h2 = jnp.dot(h1.astype(bf), w2_ref[0].astype(bf),
                 preferred_element_type=jnp.float32) + b2_ref[0]
    h2 = jnp.maximum(_ln(h2, g2_ref[0, 0], be2_ref[0, 0]), 0.0)
    out_ref[...] = jnp.dot(h2.astype(bf), w3_ref[0].astype(bf),
                           preferred_element_type=jnp.float32) + b3_ref[0]


def _ffn_call(blk, dispatch, p, ew1p):
    per_e = lambda shape: pl.BlockSpec(
        (1,) + shape, lambda i, m: (m[i], 0, 0))
    grid_spec = pltpu.PrefetchScalarGridSpec(
        num_scalar_prefetch=1,
        grid=(NBLK,),
        in_specs=[
            pl.BlockSpec((BM, DP), lambda i, m: (i, 0)),
            per_e((DP, H)), per_e((1, H)), per_e((1, H)), per_e((1, H)),
            per_e((H, H)), per_e((1, H)), per_e((1, H)), per_e((1, H)),
            per_e((H, 2 * A)), per_e((1, 2 * A)),
        ],
        out_specs=pl.BlockSpec((BM, 2 * A), lambda i, m: (i, 0)),
    )
    return pl.pallas_call(
        _ffn_body,
        grid_spec=grid_spec,
        out_shape=jax.ShapeDtypeStruct((P, 2 * A), jnp.float32),
        compiler_params=pltpu.CompilerParams(
            dimension_semantics=("arbitrary",),
        ),
    )(blk, dispatch,
      ew1p, p['exp_b1'][:, None, :], p['exp_g1'][:, None, :],
      p['exp_be1'][:, None, :],
      p['exp_w2'], p['exp_b2'][:, None, :], p['exp_g2'][:, None, :],
      p['exp_be2'][:, None, :],
      p['exp_w3'], p['exp_b3'][:, None, :])


def _sc_gather_body(ffn_hbm, pos_hbm, r0_hbm, r1_hbm, idx_v, rows_v, sem):
    wid = lax.axis_index("s") * 2 + lax.axis_index("c")
    base = wid * TPW
    for k, out in enumerate((r0_hbm, r1_hbm)):
        pltpu.sync_copy(pos_hbm.at[k, wid], idx_v)
        pltpu.async_copy(ffn_hbm.at[idx_v], rows_v, sem).wait()
        pltpu.sync_copy(rows_v, out.at[pl.ds(base, TPW)])


def _sc_gather(ffn_out, pos_sc):
    f = functools.partial(
        pl.kernel,
        out_type=(jax.ShapeDtypeStruct((B, 2 * A), jnp.float32),
                  jax.ShapeDtypeStruct((B, 2 * A), jnp.float32)),
        mesh=plsc.VectorSubcoreMesh(core_axis_name="c",
                                    subcore_axis_name="s"),
        scratch_types=[
            pltpu.VMEM((TPW,), jnp.int32),
            pltpu.VMEM((TPW, 2 * A), jnp.float32),
            pltpu.SemaphoreType.DMA,
        ],
        compiler_params=pltpu.CompilerParams(use_tc_tiling_on_sc=False),
    )(_sc_gather_body)
    return f(ffn_out, pos_sc)


def _combine_body(r0_ref, r1_ref, g0_ref, g1_ref, alpha_ref, beta_ref):
    acc = g0_ref[...] * r0_ref[...] + g1_ref[...] * r1_ref[...]
    alpha_ref[...] = jax.nn.softplus(acc[:, :A]) + 1.0
    beta_ref[...] = jax.nn.softplus(acc[:, A:]) + 1.0


def _combine_call(r0, r1, g0, g1):
    return pl.pallas_call(
        _combine_body,
        out_shape=[jax.ShapeDtypeStruct((B, A), jnp.float32),
                   jax.ShapeDtypeStruct((B, A), jnp.float32)],
    )(r0, r1, g0, g1)


def kernel(state, params):
    p = params
    state_pad = jnp.pad(state, ((0, 0), (0, DP - D)))
    ew1p = jnp.pad(p['exp_w1'], ((0, 0), (0, DP - D), (0, 0)))
    probs, g0, g1, pos, blk, value = _router_call(state, p)
    pos_sc = pos.T.reshape(2, NW, TPW)
    dispatch = _sc_dispatch(state_pad, pos_sc)
    ffn_out = _ffn_call(blk.reshape(NBLK), dispatch, p, ew1p)
    r0, r1 = _sc_gather(ffn_out, pos_sc)
    alpha, beta = _combine_call(r0, r1, g0, g1)
    return (alpha, beta, value, probs)


# speedup vs baseline: 1.2064x; 1.1163x over previous
import functools

import jax
import jax.numpy as jnp
from jax import lax
from jax.experimental import pallas as pl
from jax.experimental.pallas import tpu as pltpu
from jax.experimental.pallas import tpu_sc as plsc

B = 4096
D = 44
SP = 128
H = 1024
E = 8
A = 8
RH = H // 2
BB = 1024
NB = B // BB
BM = 256
NBLK = (B * 2) // BM + E
P = NBLK * BM
NW = 32
TPW = B // NW


def _ln(x, g, b, eps=1e-5):
    m = jnp.mean(x, axis=-1, keepdims=True)
    v = jnp.mean((x - m) * (x - m), axis=-1, keepdims=True)
    return (x - m) * jax.lax.rsqrt(v + eps) * g + b


def _meta_body(state_ref, rw1_ref, rb1_ref, rw2_ref, rb2_ref,
               probs_ref, g0_ref, g1_ref, pos_ref, blk_ref, act_ref):
    x = state_ref[...]
    h = jnp.maximum(jnp.dot(x, rw1_ref[...],
                            preferred_element_type=jnp.float32)
                    + rb1_ref[...], 0.0)
    logits = jnp.dot(h, rw2_ref[...],
                     preferred_element_type=jnp.float32) + rb2_ref[...]
    mx = jnp.max(logits, axis=-1, keepdims=True)
    ex = jnp.exp(logits - mx)
    probs = ex / jnp.sum(ex, axis=-1, keepdims=True)
    probs_ref[...] = probs
    p1 = jnp.max(probs, axis=-1, keepdims=True)
    i1 = jnp.argmax(probs, axis=-1)[:, None]
    eids = jax.lax.broadcasted_iota(jnp.int32, (B, E), 1)
    masked = jnp.where(eids == i1, -jnp.inf, probs)
    p2 = jnp.max(masked, axis=-1, keepdims=True)
    i2 = jnp.argmax(masked, axis=-1)[:, None]
    denom = p1 + p2 + 1e-8
    g0_ref[...] = p1 / denom
    g1_ref[...] = p2 / denom
    sel1 = (eids == i1).astype(jnp.float32)
    sel2 = (eids == i2).astype(jnp.float32)
    sel = sel1 + sel2
    rr = jax.lax.broadcasted_iota(jnp.int32, (512, 512), 0)
    cc = jax.lax.broadcasted_iota(jnp.int32, (512, 512), 1)
    tri = (cc < rr).astype(jnp.float32)
    carry = jnp.zeros((1, E), jnp.float32)
    chunks = []
    for c in range(B // 512):
        ch = sel[c * 512:(c + 1) * 512, :]
        chunks.append(jnp.dot(tri, ch,
                              preferred_element_type=jnp.float32) + carry)
        carry = carry + jnp.sum(ch, axis=0, keepdims=True)
    rank = jnp.concatenate(chunks, axis=0)
    counts = carry
    nblk_e = jnp.ceil(counts / BM)
    u_r = jax.lax.broadcasted_iota(jnp.int32, (E, E), 0)
    u_c = jax.lax.broadcasted_iota(jnp.int32, (E, E), 1)
    upper = (u_r < u_c).astype(jnp.float32)
    cumblk_excl = jnp.dot(nblk_e, upper,
                          preferred_element_type=jnp.float32)
    base = cumblk_excl * BM
    rank1 = jnp.sum(sel1 * rank, axis=-1, keepdims=True)
    rank2 = jnp.sum(sel2 * rank, axis=-1, keepdims=True)
    base1 = jnp.sum(sel1 * base, axis=-1, keepdims=True)
    base2 = jnp.sum(sel2 * base, axis=-1, keepdims=True)
    pos_ref[...] = jnp.concatenate(
        [base1 + rank1, base2 + rank2], axis=1).astype(jnp.int32)
    cumblk_incl = (cumblk_excl + nblk_e).astype(jnp.int32)
    biota = jax.lax.broadcasted_iota(jnp.int32, (NBLK, E), 0)
    blk = jnp.sum((biota >= cumblk_incl).astype(jnp.int32),
                  axis=-1, keepdims=True)
    blk_ref[...] = jnp.minimum(blk, E - 1)
    total = jnp.max(cumblk_incl, axis=-1, keepdims=True)
    act_ref[...] = (biota[:, :1] < total).astype(jnp.int32)


def _meta_call(state, p):
    full = lambda shape: pl.BlockSpec(shape, lambda: (0,) * len(shape))
    return pl.pallas_call(
        _meta_body,
        in_specs=[
            full((B, D)),
            full((D, RH)), full((RH,)), full((RH, E)), full((E,)),
        ],
        out_specs=[
            full((B, E)), full((B, 1)), full((B, 1)),
            full((B, 2)), full((NBLK, 1)), full((NBLK, 1)),
        ],
        out_shape=[
            jax.ShapeDtypeStruct((B, E), jnp.float32),
            jax.ShapeDtypeStruct((B, 1), jnp.float32),
            jax.ShapeDtypeStruct((B, 1), jnp.float32),
            jax.ShapeDtypeStruct((B, 2), jnp.int32),
            jax.ShapeDtypeStruct((NBLK, 1), jnp.int32),
            jax.ShapeDtypeStruct((NBLK, 1), jnp.int32),
        ],
    )(state, p['router_w1'], p['router_b1'], p['router_w2'],
      p['router_b2'])


def _value_body(xb_ref, vw1_ref, vb1_ref, vg_ref, vbe_ref,
                vw2_ref, vb2_ref, vw3_ref, vb3_ref, value_ref):
    bf = jnp.bfloat16
    xb = xb_ref[...].astype(bf)
    v = jnp.maximum(
        _ln(jnp.dot(xb, vw1_ref[...].astype(bf),
                    preferred_element_type=jnp.float32)
            + vb1_ref[...], vg_ref[...], vbe_ref[...]), 0.0)
    v = jnp.maximum(jnp.dot(v.astype(bf), vw2_ref[...].astype(bf),
                            preferred_element_type=jnp.float32)
                    + vb2_ref[...], 0.0)
    value_ref[...] = jnp.dot(v.astype(bf), vw3_ref[...].astype(bf),
                             preferred_element_type=jnp.float32) \
        + vb3_ref[...]


def _value_call(state, p):
    full = lambda shape: pl.BlockSpec(shape, lambda b: (0,) * len(shape))
    per_b = lambda shape: pl.BlockSpec(
        shape, lambda b: (b,) + (0,) * (len(shape) - 1))
    return pl.pallas_call(
        _value_body,
        grid=(NB,),
        in_specs=[
            per_b((BB, D)),
            full((D, H)), full((H,)), full((H,)), full((H,)),
            full((H, RH)), full((RH,)), full((RH, 1)), full((1,)),
        ],
        out_specs=per_b((BB, 1)),
        out_shape=jax.ShapeDtypeStruct((B, 1), jnp.float32),
        compiler_params=pltpu.CompilerParams(
            dimension_semantics=("arbitrary",),
        ),
    )(state, p['val_w1'], p['val_b1'], p['val_g'], p['val_be'],
      p['val_w2'], p['val_b2'], p['val_w3'], p['val_b3'])


def _sc_dispatch_body(state_hbm, pos_hbm, out_hbm, idx_v, rows_v, sem):
    wid = lax.axis_index("s") * 2 + lax.axis_index("c")
    base = wid * TPW
    pltpu.sync_copy(state_hbm.at[pl.ds(base, TPW)], rows_v)
    for k in range(2):
        pltpu.sync_copy(pos_hbm.at[k, wid], idx_v)
        pltpu.async_copy(rows_v, out_hbm.at[idx_v], sem).wait()


def _sc_dispatch(state_pad, pos_sc):
    f = functools.partial(
        pl.kernel,
        out_type=jax.ShapeDtypeStruct((P, SP), jnp.float32),
        mesh=plsc.VectorSubcoreMesh(core_axis_name="c",
                                    subcore_axis_name="s"),
        scratch_types=[
            pltpu.VMEM((TPW,), jnp.int32),
            pltpu.VMEM((TPW, SP), jnp.float32),
            pltpu.SemaphoreType.DMA,
        ],
        compiler_params=pltpu.CompilerParams(use_tc_tiling_on_sc=False),
    )(_sc_dispatch_body)
    return f(state_pad, pos_sc)


def _ffn_body(m_ref, a_ref, x_ref, w1_ref, b1_ref, g1_ref, be1_ref,
              w2_ref, b2_ref, g2_ref, be2_ref, w3_ref, b3_ref, out_ref):
    i = pl.program_id(0)

    @pl.when(a_ref[i] > 0)
    def _compute():
        bf = jnp.bfloat16
        x = x_ref[...].astype(bf)
        h1 = jnp.dot(x, w1_ref[0].astype(bf),
                     preferred_element_type=jnp.float32) + b1_ref[0]
        h1 = jnp.maximum(_ln(h1, g1_ref[0, 0], be1_ref[0, 0]), 0.0)
        h2 = jnp.dot(h1.astype(bf), w2_ref[0].astype(bf),
                     preferred_element_type=jnp.float32) + b2_ref[0]
        h2 = jnp.maximum(_ln(h2, g2_ref[0, 0], be2_ref[0, 0]), 0.0)
        out = jnp.dot(h2.astype(bf), w3_ref[0].astype(bf),
                      preferred_element_type=jnp.float32) + b3_ref[0]
        out_ref[...] = jnp.concatenate(
            [out, jnp.zeros((BM, SP - 2 * A), jnp.float32)], axis=1)


def _ffn_call(blk, act, dispatch, p, ew1p):
    per_e = lambda shape: pl.BlockSpec(
        (1,) + shape, lambda i, m, a: (m[i], 0, 0))
    grid_spec = pltpu.PrefetchScalarGridSpec(
        num_scalar_prefetch=2,
        grid=(NBLK,),
        in_specs=[
            pl.BlockSpec((BM, SP), lambda i, m, a: (i, 0)),
            per_e((SP, H)), per_e((1, H)), per_e((1, H)), per_e((1, H)),
            per_e((H, H)), per_e((1, H)), per_e((1, H)), per_e((1, H)),
            per_e((H, 2 * A)), per_e((1, 2 * A)),
        ],
        out_specs=pl.BlockSpec((BM, SP), lambda i, m, a: (i, 0)),
    )
    return pl.pallas_call(
        _ffn_body,
        grid_spec=grid_spec,
        out_shape=jax.ShapeDtypeStruct((P, SP), jnp.float32),
        compiler_params=pltpu.CompilerParams(
            dimension_semantics=("arbitrary",),
        ),
    )(blk, act, dispatch,
      ew1p, p['exp_b1'][:, None, :], p['exp_g1'][:, None, :],
      p['exp_be1'][:, None, :],
      p['exp_w2'], p['exp_b2'][:, None, :], p['exp_g2'][:, None, :],
      p['exp_be2'][:, None, :],
      p['exp_w3'], p['exp_b3'][:, None, :])


def _sc_gather_body(ffn_hbm, pos_hbm, r0_hbm, r1_hbm, idx_v, rows_v, sem):
    wid = lax.axis_index("s") * 2 + lax.axis_index("c")
    base = wid * TPW
    for k, out in enumerate((r0_hbm, r1_hbm)):
        pltpu.sync_copy(pos_hbm.at[k, wid], idx_v)
        pltpu.async_copy(ffn_hbm.at[idx_v], rows_v, sem).wait()
        pltpu.sync_copy(rows_v, out.at[pl.ds(base, TPW)])


def _sc_gather(ffn_out, pos_sc):
    f = functools.partial(
        pl.kernel,
        out_type=(jax.ShapeDtypeStruct((B, SP), jnp.float32),
                  jax.ShapeDtypeStruct((B, SP), jnp.float32)),
        mesh=plsc.VectorSubcoreMesh(core_axis_name="c",
                                    subcore_axis_name="s"),
        scratch_types=[
            pltpu.VMEM((TPW,), jnp.int32),
            pltpu.VMEM((TPW, SP), jnp.float32),
            pltpu.SemaphoreType.DMA,
        ],
        compiler_params=pltpu.CompilerParams(use_tc_tiling_on_sc=False),
    )(_sc_gather_body)
    return f(ffn_out, pos_sc)


def _combine_body(r0_ref, r1_ref, g0_ref, g1_ref, alpha_ref, beta_ref):
    acc = g0_ref[...] * r0_ref[:, :2 * A] + g1_ref[...] * r1_ref[:, :2 * A]
    alpha_ref[...] = jax.nn.softplus(acc[:, :A]) + 1.0
    beta_ref[...] = jax.nn.softplus(acc[:, A:]) + 1.0


def _combine_call(r0, r1, g0, g1):
    per_b = lambda shape: pl.BlockSpec(
        shape, lambda b: (b,) + (0,) * (len(shape) - 1))
    return pl.pallas_call(
        _combine_body,
        grid=(NB,),
        in_specs=[per_b((BB, SP)), per_b((BB, SP)),
                  per_b((BB, 1)), per_b((BB, 1))],
        out_specs=[per_b((BB, A)), per_b((BB, A))],
        out_shape=[jax.ShapeDtypeStruct((B, A), jnp.float32),
                   jax.ShapeDtypeStruct((B, A), jnp.float32)],
        compiler_params=pltpu.CompilerParams(
            dimension_semantics=("arbitrary",),
        ),
    )(r0, r1, g0, g1)


def kernel(state, params):
    p = params
    state_pad = jnp.pad(state, ((0, 0), (0, SP - D)))
    ew1p = jnp.pad(p['exp_w1'], ((0, 0), (0, SP - D), (0, 0)))
    probs, g0, g1, pos, blk, act = _meta_call(state, p)
    value = _value_call(state, p)
    pos_sc = pos.T.reshape(2, NW, TPW)
    dispatch = _sc_dispatch(state_pad, pos_sc)
    ffn_out = _ffn_call(blk.reshape(NBLK), act.reshape(NBLK),
                        dispatch, p, ew1p)
    r0, r1 = _sc_gather(ffn_out, pos_sc)
    alpha, beta = _combine_call(r0, r1, g0, g1)
    return (alpha, beta, value, probs)
